# baseline (speedup 1.0000x reference)
import jax
import jax.numpy as jnp
from jax import lax
from jax.experimental import pallas as pl
from jax.experimental.pallas import tpu as pltpu
from jax.experimental.pallas import tpu_sc as plsc

B = 16384
H = 50
D = 32
L = 16
NW = 32
BPW = B // NW
GR = 4
GW = GR * H
NG = BPW // GR
NBUF = 4
SPLIT = 104

_DNUMS = lax.GatherDimensionNumbers(
    offset_dims=(), collapsed_slice_dims=(0,), start_index_map=(0,))


def _splat(vec, lane_idx):
    return lax.gather(vec, lane_idx, _DNUMS, (1,),
                      mode=lax.GatherScatterMode.PROMISE_IN_BOUNDS)


def _sc_body(table_hbm, idx_hbm, mask_hbm, out_hbm,
             idx_v, mask_v, rows_v, out_v, *sems):
    wid = lax.axis_index("s") * 2 + lax.axis_index("c")
    base_w = wid * (BPW * H)

    pltpu.sync_copy(idx_hbm.at[pl.ds(base_w, BPW * H)],
                    idx_v.at[pl.ds(0, BPW * H)])
    pltpu.sync_copy(mask_hbm.at[pl.ds(base_w, BPW * H)],
                    mask_v.at[pl.ds(0, BPW * H)])

    lane_consts = [jnp.full((L, 1), i, jnp.int32) for i in range(L)]

    def issue(g, b):
        off = pl.multiple_of(g * GW, 8)
        pltpu.make_async_copy(table_hbm.at[idx_v.at[pl.ds(off, SPLIT)]],
                              rows_v.at[b, pl.ds(0, SPLIT)],
                              sems[2 * b]).start()
        off2 = pl.multiple_of(g * GW + SPLIT, 8)
        pltpu.make_async_copy(table_hbm.at[idx_v.at[pl.ds(off2, GW - SPLIT)]],
                              rows_v.at[b, pl.ds(SPLIT, GW - SPLIT)],
                              sems[2 * b + 1]).start()

    def drain(b):
        pltpu.make_async_copy(table_hbm.at[idx_v.at[pl.ds(0, SPLIT)]],
                              rows_v.at[b, pl.ds(0, SPLIT)],
                              sems[2 * b]).wait()
        pltpu.make_async_copy(table_hbm.at[idx_v.at[pl.ds(0, GW - SPLIT)]],
                              rows_v.at[b, pl.ds(SPLIT, GW - SPLIT)],
                              sems[2 * b + 1]).wait()

    for b in range(NBUF):
        issue(b, b)

    nvec = (GW + L - 1) // L

    def group_compute(g, b):
        moff = pl.multiple_of(g * GW, 8)
        mv = [mask_v[pl.ds(moff + L * k, L)] for k in range(nvec)]
        for r in range(GR):
            acc0 = jnp.zeros((L,), jnp.float32)
            acc1 = jnp.zeros((L,), jnp.float32)
            den = jnp.zeros((L,), jnp.float32)
            for l in range(H):
                w = r * H + l
                m = _splat(mv[w // L], lane_consts[w % L])
                den = den + m
                acc0 = acc0 + m * rows_v[b, w, pl.ds(0, L)]
                acc1 = acc1 + m * rows_v[b, w, pl.ds(L, L)]
            inv = jnp.ones((L,), jnp.float32) / jnp.maximum(den, 1.0)
            orow = GR * g + r
            out_v[orow, pl.ds(0, L)] = acc0 * inv
            out_v[orow, pl.ds(L, L)] = acc1 * inv

    def g_body(gg, carry):
        for b in range(NBUF):
            g = gg * NBUF + b
            drain(b)
            group_compute(g, b)
            ng = g + NBUF

            @pl.when(ng < NG)
            def _():
                issue(ng, b)
        return carry

    lax.fori_loop(0, NG // NBUF, g_body, 0)
    pltpu.sync_copy(out_v, out_hbm.at[pl.ds(wid * BPW, BPW)])


@jax.jit
def _run(table, idx, mask):
    mesh = plsc.VectorSubcoreMesh(core_axis_name="c", subcore_axis_name="s")
    f = pl.kernel(
        _sc_body,
        out_type=jax.ShapeDtypeStruct((B, D), jnp.float32),
        mesh=mesh,
        compiler_params=pltpu.CompilerParams(use_tc_tiling_on_sc=False),
        scratch_types=[
            pltpu.VMEM((BPW * H + L,), jnp.int32),
            pltpu.VMEM((BPW * H + L,), jnp.float32),
            pltpu.VMEM((NBUF, GW, D), jnp.float32),
            pltpu.VMEM((BPW, D), jnp.float32),
        ] + [pltpu.SemaphoreType.DMA] * (2 * NBUF),
    )
    return f(table, idx, mask)


TBLK = 8192


def _tp_body(in_ref, out_ref):
    out_ref[...] = in_ref[...].T


def _transpose_table(table_t):
    n = table_t.shape[1]
    grid = (n + TBLK - 1) // TBLK
    return pl.pallas_call(
        _tp_body,
        grid=(grid,),
        in_specs=[pl.BlockSpec((D, TBLK), lambda i: (0, i))],
        out_specs=pl.BlockSpec((TBLK, D), lambda i: (i, 0)),
        out_shape=jax.ShapeDtypeStruct((grid * TBLK, D), jnp.float32),
        compiler_params=pltpu.CompilerParams(
            dimension_semantics=("arbitrary",)),
    )(table_t)


def kernel(ingredients, mask, table):
    idx1 = jnp.maximum(ingredients.reshape(-1), 0)
    mask1 = jnp.maximum(mask.reshape(-1), -1.0)
    table1 = _transpose_table(table.T)
    return _run(table1, idx1, mask1)


# speedup vs baseline: 1.7762x; 1.7762x over previous
import jax
import jax.numpy as jnp
from jax import lax
from jax.experimental import pallas as pl
from jax.experimental.pallas import tpu as pltpu
from jax.experimental.pallas import tpu_sc as plsc

B = 16384
H = 50
D = 32
L = 16
NW = 32
BPW = B // NW
GR = 4
GW = GR * H
NG = BPW // GR
NBUF = 4
SPLIT = 104

_DNUMS = lax.GatherDimensionNumbers(
    offset_dims=(), collapsed_slice_dims=(0,), start_index_map=(0,))


def _splat(vec, lane_idx):
    return lax.gather(vec, lane_idx, _DNUMS, (1,),
                      mode=lax.GatherScatterMode.PROMISE_IN_BOUNDS)


def _sc_body(table_hbm, idx_hbm, mask_hbm, out_hbm,
             idx_v, mask_v, rows_v, out_v, *sems):
    wid = lax.axis_index("s") * 2 + lax.axis_index("c")
    base_w = wid * (BPW * H)

    pltpu.sync_copy(idx_hbm.at[pl.ds(base_w, BPW * H)],
                    idx_v.at[pl.ds(0, BPW * H)])
    pltpu.sync_copy(mask_hbm.at[pl.ds(base_w, BPW * H)],
                    mask_v.at[pl.ds(0, BPW * H)])

    lane_consts = [jnp.full((L, 1), i, jnp.int32) for i in range(L)]

    def issue(g, b):
        off = pl.multiple_of(g * GW, 8)
        pltpu.make_async_copy(table_hbm.at[idx_v.at[pl.ds(off, SPLIT)]],
                              rows_v.at[b, pl.ds(0, SPLIT)],
                              sems[2 * b]).start()
        off2 = pl.multiple_of(g * GW + SPLIT, 8)
        pltpu.make_async_copy(table_hbm.at[idx_v.at[pl.ds(off2, GW - SPLIT)]],
                              rows_v.at[b, pl.ds(SPLIT, GW - SPLIT)],
                              sems[2 * b + 1]).start()

    def drain(b):
        pltpu.make_async_copy(table_hbm.at[idx_v.at[pl.ds(0, SPLIT)]],
                              rows_v.at[b, pl.ds(0, SPLIT)],
                              sems[2 * b]).wait()
        pltpu.make_async_copy(table_hbm.at[idx_v.at[pl.ds(0, GW - SPLIT)]],
                              rows_v.at[b, pl.ds(SPLIT, GW - SPLIT)],
                              sems[2 * b + 1]).wait()

    for b in range(NBUF):
        issue(b, b)

    nvec = (GW + L - 1) // L

    def group_compute(g, b):
        moff = pl.multiple_of(g * GW, 8)
        mv = [mask_v[pl.ds(moff + L * k, L)] for k in range(nvec)]
        for r in range(GR):
            acc0 = jnp.zeros((L,), jnp.float32)
            acc1 = jnp.zeros((L,), jnp.float32)
            den = jnp.zeros((L,), jnp.float32)
            for l in range(H):
                w = r * H + l
                m = _splat(mv[w // L], lane_consts[w % L])
                den = den + m
                acc0 = acc0 + m * rows_v[b, w, pl.ds(0, L)]
                acc1 = acc1 + m * rows_v[b, w, pl.ds(L, L)]
            inv = jnp.ones((L,), jnp.float32) / jnp.maximum(den, 1.0)
            orow = GR * g + r
            out_v[orow, pl.ds(0, L)] = acc0 * inv
            out_v[orow, pl.ds(L, L)] = acc1 * inv

    def g_body(gg, carry):
        for b in range(NBUF):
            g = gg * NBUF + b
            drain(b)
            group_compute(g, b)
            ng = g + NBUF

            @pl.when(ng < NG)
            def _():
                issue(ng, b)
        return carry

    lax.fori_loop(0, NG // NBUF, g_body, 0)
    pltpu.sync_copy(out_v, out_hbm.at[pl.ds(wid * BPW, BPW)])


@jax.jit
def _run(table, idx, mask):
    mesh = plsc.VectorSubcoreMesh(core_axis_name="c", subcore_axis_name="s")
    f = pl.kernel(
        _sc_body,
        out_type=jax.ShapeDtypeStruct((B, D), jnp.float32),
        mesh=mesh,
        compiler_params=pltpu.CompilerParams(use_tc_tiling_on_sc=False),
        scratch_types=[
            pltpu.VMEM((BPW * H + L,), jnp.int32),
            pltpu.VMEM((BPW * H + L,), jnp.float32),
            pltpu.VMEM((NBUF, GW, D), jnp.float32),
            pltpu.VMEM((BPW, D), jnp.float32),
        ] + [pltpu.SemaphoreType.DMA] * (2 * NBUF),
    )
    return f(table, idx, mask)


TBLK = 8192
QLOG = 18
Q = 1 << QLOG
TGRID = Q // TBLK


def _tp_body(r0, r1, r2, r3, out_ref):
    out_ref[...] = jnp.concatenate(
        [r0[...].T, r1[...].T, r2[...].T, r3[...].T], axis=1)


def _transpose_table(table_t):
    kmax = (table_t.shape[1] - 1) // TBLK
    specs = [
        pl.BlockSpec((D, TBLK),
                     lambda i, k=k: (0, jnp.minimum(i + k * TGRID, kmax)))
        for k in range(4)
    ]
    out = pl.pallas_call(
        _tp_body,
        grid=(TGRID,),
        in_specs=specs,
        out_specs=pl.BlockSpec((TBLK, 4 * D), lambda i: (i, 0)),
        out_shape=jax.ShapeDtypeStruct((Q, 4 * D), jnp.float32),
        compiler_params=pltpu.CompilerParams(
            dimension_semantics=("arbitrary",)),
    )(table_t, table_t, table_t, table_t)
    return out.reshape(4 * Q, D)


def kernel(ingredients, mask, table):
    idx0 = jnp.maximum(ingredients.reshape(-1), 0)
    idx1 = 4 * (idx0 & (Q - 1)) + (idx0 >> QLOG)
    mask1 = jnp.maximum(mask.reshape(-1), -1.0)
    table1 = _transpose_table(table.T)
    return _run(table1, idx1, mask1)
